# Initial kernel scaffold; baseline (speedup 1.0000x reference)
import jax
import jax.numpy as jnp
from jax.experimental import pallas as pl


def kernel(x, edge_index, W1l, W1r, att1, b1, W2l, W2r, att2, b2):
    raise NotImplementedError("write your pallas kernel here")


# speedup vs baseline: 1.0115x; 1.0115x over previous
import functools

import jax
import jax.numpy as jnp
from jax.experimental import pallas as pl
from jax.experimental.pallas import tpu as pltpu

N = 10000
H1 = 8
C1 = 128
C2 = 64


def _mm_kernel(x_ref, w_ref, o_ref):
    o_ref[...] = jnp.dot(x_ref[...], w_ref[...],
                         preferred_element_type=jnp.float32)


def _matmul(x, w, block_m=1024):
    m, k = x.shape
    _, n = w.shape
    pad_m = (-m) % block_m
    xp = jnp.pad(x, ((0, pad_m), (0, 0)))
    out = pl.pallas_call(
        _mm_kernel,
        grid=((m + pad_m) // block_m,),
        in_specs=[
            pl.BlockSpec((block_m, k), lambda i: (i, 0)),
            pl.BlockSpec((k, n), lambda i: (0, 0)),
        ],
        out_specs=pl.BlockSpec((block_m, n), lambda i: (i, 0)),
        out_shape=jax.ShapeDtypeStruct((m + pad_m, n), jnp.float32),
    )(xp, w)
    return out[:m]


def _gatv2(x, src, dst, Wl, Wr, att, bias, heads, out_ch):
    n = x.shape[0]
    xl = _matmul(x, Wl).reshape(n, heads, out_ch)
    xr = _matmul(x, Wr).reshape(n, heads, out_ch)
    e = xl[src] + xr[dst]
    e = jax.nn.leaky_relu(e, negative_slope=0.2)
    alpha = jnp.sum(e * att[None, :, :], axis=-1)
    amax = jax.ops.segment_max(alpha, dst, num_segments=n)
    alpha = jnp.exp(alpha - amax[dst])
    denom = jax.ops.segment_sum(alpha, dst, num_segments=n)
    alpha = alpha / (denom[dst] + 1e-16)
    msg = xl[src] * alpha[:, :, None]
    out = jax.ops.segment_sum(msg, dst, num_segments=n)
    return out.reshape(n, heads * out_ch) + bias


def kernel(x, edge_index, W1l, W1r, att1, b1, W2l, W2r, att2, b2):
    n = x.shape[0]
    loop = jnp.arange(n, dtype=edge_index.dtype)
    src = jnp.concatenate([edge_index[0], loop])
    dst = jnp.concatenate([edge_index[1], loop])
    h = _gatv2(x, src, dst, W1l, W1r, att1, b1, H1, C1)
    h = jax.nn.relu(h)
    h = _gatv2(h, src, dst, W2l, W2r, att2, b2, 1, C2)
    return jax.nn.log_softmax(h, axis=1)
